# Initial kernel scaffold; baseline (speedup 1.0000x reference)
import jax
import jax.numpy as jnp
from jax.experimental import pallas as pl


def kernel(p, q, x, W0, b0, g0, be0, W1, b1, g1, be1, W2, b2, g2, be2):
    raise NotImplementedError("write your pallas kernel here")


# speedup vs baseline: 24.7581x; 24.7581x over previous
import functools
import numpy as np
import jax
import jax.numpy as jnp
from jax import lax
from jax.experimental import pallas as pl
from jax.experimental.pallas import tpu as pltpu
from jax.experimental.pallas import tpu_sc as plsc

_B, _N, _M, _C, _K = 4, 8192, 2048, 64, 32
_R2 = np.float32(np.float64(0.4) * np.float64(0.4))
_EPS = np.float32(1e-5)
_P = _B * _M * _K

_MB = 256
_CN = 512
_NCH = _N // _CN


def _ballq_body(q_ref, pt_ref, idx_ref, cnt_ref, slots_ref):
    b = pl.program_id(0)
    cnt_ref[...] = jnp.zeros((_MB, 1), jnp.float32)
    slots_ref[...] = jnp.zeros((_MB, _K), jnp.float32)
    qb = q_ref[0]
    q2 = jnp.sum(qb * qb, axis=1, keepdims=True)
    rowi = lax.broadcasted_iota(jnp.int32, (_CN, _CN), 0)
    coli = lax.broadcasted_iota(jnp.int32, (_CN, _CN), 1)
    tri = jnp.where(rowi <= coli, 1.0, 0.0).astype(jnp.float32)
    lane = lax.broadcasted_iota(jnp.int32, (_MB, _CN), 1).astype(jnp.float32)
    kcol = lax.broadcasted_iota(jnp.int32, (_MB, _K), 1).astype(jnp.float32)

    def chunk(c, carry):
        @pl.when(jnp.min(cnt_ref[...]) < float(_K))
        def _():
            cnt = cnt_ref[...]
            pc = pt_ref[0, c]
            p2 = jnp.sum(pc * pc, axis=0, keepdims=True)
            qp = lax.dot_general(qb, pc, (((1,), (0,)), ((), ())),
                                 preferred_element_type=jnp.float32)
            d2 = (q2 + p2) - 2.0 * qp
            maskf = jnp.where(d2 <= _R2, 1.0, 0.0).astype(jnp.float32)
            r = jnp.dot(maskf, tri, preferred_element_type=jnp.float32) + cnt
            nval = (lane + (c * _CN + 1).astype(jnp.float32)) * maskf

            def kstep(k, slots):
                kf = (k + 1).astype(jnp.float32)
                cand = jnp.where(r == kf, nval, 0.0)
                v = jnp.max(cand, axis=1, keepdims=True)
                return jnp.maximum(slots, jnp.where(kcol == kf - 1.0, v, 0.0))

            slots_ref[...] = lax.fori_loop(0, _K, kstep, slots_ref[...])
            cnt_ref[...] = cnt + jnp.sum(maskf, axis=1, keepdims=True)
        return carry

    lax.fori_loop(0, _NCH, chunk, 0)
    slots = slots_ref[...]
    first = slots[:, 0:1]
    filled = jnp.where(slots > 0.0, slots, jnp.maximum(first, 1.0))
    idx_ref[0] = (filled - 1.0).astype(jnp.int32) + b * _N


def _ball_query(q, pt_r):
    return pl.pallas_call(
        _ballq_body,
        grid=(_B, _M // _MB),
        in_specs=[
            pl.BlockSpec((1, _MB, 3), lambda b, m: (b, m, 0)),
            pl.BlockSpec((1, _NCH, 3, _CN), lambda b, m: (b, 0, 0, 0)),
        ],
        out_specs=pl.BlockSpec((1, _MB, _K), lambda b, m: (b, m, 0)),
        out_shape=jax.ShapeDtypeStruct((_B, _M, _K), jnp.int32),
        scratch_shapes=[
            pltpu.VMEM((_MB, 1), jnp.float32),
            pltpu.VMEM((_MB, _K), jnp.float32),
        ],
    )(q, pt_r)


_DROW = 128
_NW = 32
_RPW = _P // _NW
_GCH = 128
_NJ = _RPW // _GCH


def _sc_gather(table, idx3):
    mesh = plsc.VectorSubcoreMesh(core_axis_name="c", subcore_axis_name="s")

    @functools.partial(
        pl.kernel,
        out_type=jax.ShapeDtypeStruct((_P, _DROW), jnp.float32),
        mesh=mesh,
        scratch_types=[
            pltpu.VMEM((_NJ, _GCH), jnp.int32),
            pltpu.VMEM((_GCH, _DROW), jnp.float32),
            pltpu.VMEM((_GCH, _DROW), jnp.float32),
            pltpu.SemaphoreType.DMA,
            pltpu.SemaphoreType.DMA,
        ],
    )
    def gather_k(table_hbm, idx_hbm, out_hbm, idx_v, rows_a, rows_b, sem_a, sem_b):
        wid = lax.axis_index("s") * 2 + lax.axis_index("c")
        base = wid * _RPW
        pltpu.sync_copy(idx_hbm.at[wid], idx_v)
        bufs = (rows_a, rows_b)
        sems = (sem_a, sem_b)

        def step(j, _):
            for t in range(2):
                jj = j + t
                cp = pltpu.async_copy(
                    table_hbm.at[idx_v.at[jj]], bufs[t], sems[t])
                cp.wait()
                pltpu.sync_copy(
                    bufs[t], out_hbm.at[pl.ds(base + jj * _GCH, _GCH)])
            return 0

        lax.fori_loop(0, _NJ // 2, lambda j, c: step(j * 2, c), 0)

    return gather_k(table, idx3)


_QB = 64
_RB = _QB * _K


def _l0_body(g_ref, qp_ref, w_ref, b_ref, h_ref, s_ref, ss_ref):
    @pl.when(pl.program_id(0) == 0)
    def _():
        s_ref[...] = jnp.zeros_like(s_ref)
        ss_ref[...] = jnp.zeros_like(ss_ref)
    xh = g_ref[...] - qp_ref[...]
    xh = xh.reshape(_RB, _DROW)
    h = jnp.dot(xh, w_ref[...], preferred_element_type=jnp.float32) + b_ref[...]
    h_ref[...] = h
    s_ref[...] += jnp.sum(h, axis=0, keepdims=True)
    ss_ref[...] += jnp.sum(h * h, axis=0, keepdims=True)


def _layer0(g3, qpad, w0c, b0):
    nblk = (_B * _M) // _QB
    return pl.pallas_call(
        _l0_body,
        grid=(nblk,),
        in_specs=[
            pl.BlockSpec((_QB, _K, _DROW), lambda i: (i, 0, 0)),
            pl.BlockSpec((_QB, 1, _DROW), lambda i: (i, 0, 0)),
            pl.BlockSpec((_DROW, _C), lambda i: (0, 0)),
            pl.BlockSpec((1, _C), lambda i: (0, 0)),
        ],
        out_specs=[
            pl.BlockSpec((_RB, _C), lambda i: (i, 0)),
            pl.BlockSpec((1, _C), lambda i: (0, 0)),
            pl.BlockSpec((1, _C), lambda i: (0, 0)),
        ],
        out_shape=[
            jax.ShapeDtypeStruct((_P, _C), jnp.float32),
            jax.ShapeDtypeStruct((1, _C), jnp.float32),
            jax.ShapeDtypeStruct((1, _C), jnp.float32),
        ],
    )(g3, qpad, w0c, b0)


def _l1_body(h_ref, sc_ref, sh_ref, w_ref, b_ref, o_ref, s_ref, ss_ref):
    @pl.when(pl.program_id(0) == 0)
    def _():
        s_ref[...] = jnp.zeros_like(s_ref)
        ss_ref[...] = jnp.zeros_like(ss_ref)
    a = jnp.maximum(h_ref[...] * sc_ref[...] + sh_ref[...], 0.0)
    h = jnp.dot(a, w_ref[...], preferred_element_type=jnp.float32) + b_ref[...]
    o_ref[...] = h
    s_ref[...] += jnp.sum(h, axis=0, keepdims=True)
    ss_ref[...] += jnp.sum(h * h, axis=0, keepdims=True)


def _layer1(h0, scale0, shift0, w1t, b1):
    nblk = _P // _RB
    return pl.pallas_call(
        _l1_body,
        grid=(nblk,),
        in_specs=[
            pl.BlockSpec((_RB, _C), lambda i: (i, 0)),
            pl.BlockSpec((1, _C), lambda i: (0, 0)),
            pl.BlockSpec((1, _C), lambda i: (0, 0)),
            pl.BlockSpec((_C, _C), lambda i: (0, 0)),
            pl.BlockSpec((1, _C), lambda i: (0, 0)),
        ],
        out_specs=[
            pl.BlockSpec((_RB, _C), lambda i: (i, 0)),
            pl.BlockSpec((1, _C), lambda i: (0, 0)),
            pl.BlockSpec((1, _C), lambda i: (0, 0)),
        ],
        out_shape=[
            jax.ShapeDtypeStruct((_P, _C), jnp.float32),
            jax.ShapeDtypeStruct((1, _C), jnp.float32),
            jax.ShapeDtypeStruct((1, _C), jnp.float32),
        ],
    )(h0, scale0, shift0, w1t, b1)


_C2 = 128


def _l2_body(h_ref, sc_ref, sh_ref, w_ref, b_ref,
             mx_ref, mn_ref, s_ref, ss_ref):
    @pl.when(pl.program_id(0) == 0)
    def _():
        s_ref[...] = jnp.zeros_like(s_ref)
        ss_ref[...] = jnp.zeros_like(ss_ref)
    a = jnp.maximum(h_ref[...] * sc_ref[...] + sh_ref[...], 0.0)
    h = jnp.dot(a, w_ref[...], preferred_element_type=jnp.float32) + b_ref[...]
    s_ref[...] += jnp.sum(h, axis=0, keepdims=True)
    ss_ref[...] += jnp.sum(h * h, axis=0, keepdims=True)
    h3 = h.reshape(_QB, _K, _C2)
    mx_ref[...] = jnp.max(h3, axis=1)
    mn_ref[...] = jnp.min(h3, axis=1)


def _layer2(h1, scale1, shift1, w2t, b2):
    nblk = _P // _RB
    bm = _B * _M
    return pl.pallas_call(
        _l2_body,
        grid=(nblk,),
        in_specs=[
            pl.BlockSpec((_RB, _C), lambda i: (i, 0)),
            pl.BlockSpec((1, _C), lambda i: (0, 0)),
            pl.BlockSpec((1, _C), lambda i: (0, 0)),
            pl.BlockSpec((_C, _C2), lambda i: (0, 0)),
            pl.BlockSpec((1, _C2), lambda i: (0, 0)),
        ],
        out_specs=[
            pl.BlockSpec((_QB, _C2), lambda i: (i, 0)),
            pl.BlockSpec((_QB, _C2), lambda i: (i, 0)),
            pl.BlockSpec((1, _C2), lambda i: (0, 0)),
            pl.BlockSpec((1, _C2), lambda i: (0, 0)),
        ],
        out_shape=[
            jax.ShapeDtypeStruct((bm, _C2), jnp.float32),
            jax.ShapeDtypeStruct((bm, _C2), jnp.float32),
            jax.ShapeDtypeStruct((1, _C2), jnp.float32),
            jax.ShapeDtypeStruct((1, _C2), jnp.float32),
        ],
    )(h1, scale1, shift1, w2t, b2)


def _fin_body(mx_ref, mn_ref, sc_ref, sh_ref, o_ref):
    sc = sc_ref[...]
    ext = jnp.where(sc >= 0.0, mx_ref[...], mn_ref[...])
    o_ref[...] = jnp.maximum(ext * sc + sh_ref[...], 0.0)


def _final(mx, mn, scale2, shift2):
    bm = _B * _M
    rb = 512
    return pl.pallas_call(
        _fin_body,
        grid=(bm // rb,),
        in_specs=[
            pl.BlockSpec((rb, _C2), lambda i: (i, 0)),
            pl.BlockSpec((rb, _C2), lambda i: (i, 0)),
            pl.BlockSpec((1, _C2), lambda i: (0, 0)),
            pl.BlockSpec((1, _C2), lambda i: (0, 0)),
        ],
        out_specs=pl.BlockSpec((rb, _C2), lambda i: (i, 0)),
        out_shape=jax.ShapeDtypeStruct((bm, _C2), jnp.float32),
    )(mx, mn, scale2, shift2)


def _bn_coeffs(s, ss, g, be):
    mean = s / float(_P)
    var = ss / float(_P) - mean * mean
    inv = 1.0 / jnp.sqrt(var + _EPS)
    scale = g[None, :] * inv
    shift = be[None, :] - mean * scale
    return scale, shift


def kernel(p, q, x, W0, b0, g0, be0, W1, b1, g1, be1, W2, b2, g2, be2):
    pt = jnp.transpose(p, (0, 2, 1))
    pt_r = pt.reshape(_B, 3, _NCH, _CN).transpose(0, 2, 1, 3)
    idx = _ball_query(q, pt_r)
    idx3 = idx.reshape(_NW, _NJ, _GCH)

    ppad = jnp.pad(p, ((0, 0), (0, 0), (0, 13)))
    xt = jnp.transpose(x, (0, 2, 1))
    table = jnp.concatenate(
        [ppad, xt, jnp.zeros((_B, _N, _DROW - 16 - _C), jnp.float32)],
        axis=2).reshape(_B * _N, _DROW)

    g = _sc_gather(table, idx3)

    g3 = g.reshape(_B * _M, _K, _DROW)
    qpad = jnp.pad(q, ((0, 0), (0, 0), (0, _DROW - 3)))
    qpad = qpad.reshape(_B * _M, 1, _DROW)

    w0c = jnp.zeros((_DROW, _C), jnp.float32)
    w0c = w0c.at[0:3].set(W0[:, 0:3].T).at[16:16 + _C].set(W0[:, 3:].T)

    h0, s0, ss0 = _layer0(g3, qpad, w0c, b0[None, :])
    scale0, shift0 = _bn_coeffs(s0, ss0, g0, be0)
    h1, s1, ss1 = _layer1(h0, scale0, shift0, W1.T, b1[None, :])
    scale1, shift1 = _bn_coeffs(s1, ss1, g1, be1)
    mx, mn, s2, ss2 = _layer2(h1, scale1, shift1, W2.T, b2[None, :])
    scale2, shift2 = _bn_coeffs(s2, ss2, g2, be2)
    out_bm = _final(mx, mn, scale2, shift2)
    out = out_bm.reshape(_B, _M, _C2).transpose(0, 2, 1)
    return (q, out)
